# baseline (speedup 1.0000x reference)
import jax
import jax.numpy as jnp
from jax.experimental import pallas as pl

_CBLK = 8
_LANES = 128


def _shift_kernel(ls_ref, dr_ref, x_ref, o_ref):
    x2 = x_ref[0]
    ls = ls_ref[...]
    dr = dr_ref[...]
    up = jnp.roll(x2, -1, axis=0)
    dn = jnp.roll(x2, 1, axis=0)
    g0 = jnp.take_along_axis(x2, ls, axis=-1)
    gp = jnp.take_along_axis(up, ls, axis=-1)
    gm = jnp.take_along_axis(dn, ls, axis=-1)
    o_ref[0] = jnp.where(dr == 0, g0, jnp.where(dr > 0, gp, gm))


def kernel(x, shift_indices):
    B, C, T, V = x.shape
    n = T * V
    rows = n // _LANES
    cblk = _CBLK

    si = shift_indices.astype(jnp.int32)
    i = jnp.arange(n, dtype=jnp.int32)
    j = 25 * (i // V)[None, :] + si[:, (i % V)]
    r_out = (i // _LANES)[None, :]
    dr = (j // _LANES) - r_out
    ls = j % _LANES
    dr = dr.reshape(C * rows, _LANES)
    ls = ls.reshape(C * rows, _LANES)

    xf = x.reshape(B, C * rows, _LANES)
    grid = (C // cblk, B)
    out = pl.pallas_call(
        _shift_kernel,
        grid=grid,
        in_specs=[
            pl.BlockSpec((cblk * rows, _LANES), lambda jdx, b: (jdx, 0)),
            pl.BlockSpec((cblk * rows, _LANES), lambda jdx, b: (jdx, 0)),
            pl.BlockSpec((1, cblk * rows, _LANES), lambda jdx, b: (b, jdx, 0)),
        ],
        out_specs=pl.BlockSpec((1, cblk * rows, _LANES), lambda jdx, b: (b, jdx, 0)),
        out_shape=jax.ShapeDtypeStruct((B, C * rows, _LANES), x.dtype),
    )(ls, dr, xf)
    return out.reshape(B, C, T, V)


# speedup vs baseline: 1.8889x; 1.8889x over previous
import jax
import jax.numpy as jnp
from jax.experimental import pallas as pl

_CBLK = 32


def _shift_kernel(idx_ref, x_ref, o_ref):
    xv = x_ref[0]
    idx = idx_ref[...].astype(jnp.int32)
    idxb = jnp.broadcast_to(idx[:, None, :], xv.shape)
    o_ref[0] = jnp.take_along_axis(xv, idxb, axis=-1)


def kernel(x, shift_indices):
    B, C, T, V = x.shape
    cblk = _CBLK
    grid = (B, C // cblk)
    return pl.pallas_call(
        _shift_kernel,
        grid=grid,
        in_specs=[
            pl.BlockSpec((cblk, V), lambda b, j: (j, 0)),
            pl.BlockSpec((1, cblk, T, V), lambda b, j: (b, j, 0, 0)),
        ],
        out_specs=pl.BlockSpec((1, cblk, T, V), lambda b, j: (b, j, 0, 0)),
        out_shape=jax.ShapeDtypeStruct((B, C, T, V), x.dtype),
    )(shift_indices.astype(jnp.int32), x)
